# Initial kernel scaffold; baseline (speedup 1.0000x reference)
import jax
import jax.numpy as jnp
from jax.experimental import pallas as pl


def kernel(x_user, x_item, edge_index, W_proj_user, b_proj_user, W_proj_item, b_proj_item, W_score, b_score, anchors):
    raise NotImplementedError("write your pallas kernel here")


# speedup vs baseline: 5.2805x; 5.2805x over previous
import functools

import jax
import jax.numpy as jnp
from jax import lax
from jax.experimental import pallas as pl
from jax.experimental.pallas import tpu as pltpu
from jax.experimental.pallas import tpu_sc as plsc

_D = 128
_K = 16


def _tables_body(xu_ref, xi_ref, wpu_ref, bpu_ref, wpi_ref, bpi_ref,
                 wsu_ref, wsi_ref, bs_ref, lt_ref):
    dn = (((0,), (1,)), ((), ()))
    wu = jnp.dot(wpu_ref[:], wsu_ref[:], preferred_element_type=jnp.float32, precision=lax.Precision.HIGHEST)
    cu = jnp.dot(bpu_ref[:], wsu_ref[:], preferred_element_type=jnp.float32, precision=lax.Precision.HIGHEST)
    lt_ref[0:_K, :] = (lax.dot_general(wu, xu_ref[:], dn,
                                       preferred_element_type=jnp.float32, precision=lax.Precision.HIGHEST)
                       + cu.reshape(_K, 1))
    wi = jnp.dot(wpi_ref[:], wsi_ref[:], preferred_element_type=jnp.float32, precision=lax.Precision.HIGHEST)
    ci = (jnp.dot(bpi_ref[:], wsi_ref[:], preferred_element_type=jnp.float32, precision=lax.Precision.HIGHEST)
          + bs_ref[:])
    lt_ref[_K:2 * _K, :] = (lax.dot_general(wi, xi_ref[:], dn,
                                            preferred_element_type=jnp.float32, precision=lax.Precision.HIGHEST)
                            + ci.reshape(_K, 1))


def _compute_tables(x_user, x_item, wpu, bpu, wpi, bpi, wsu, wsi, bs):
    n = x_user.shape[0]
    assert x_item.shape[0] == n
    return pl.pallas_call(
        _tables_body,
        out_shape=jax.ShapeDtypeStruct((2 * _K, n), jnp.float32),
    )(x_user, x_item, wpu, bpu.reshape(1, _D), wpi, bpi.reshape(1, _D),
      wsu, wsi, bs.reshape(1, _K))


_NG = 8
_KH = _K // 2


def _make_sc_gather(n, E, C):
    info = plsc.get_sparse_core_info()
    NC, NS = info.num_cores, info.num_subcores
    assert NC * NS == 32 and C % 128 == 0 and E % C == 0
    n_chunks = E // C
    mesh = plsc.VectorSubcoreMesh(core_axis_name="c", subcore_axis_name="s")

    @functools.partial(
        pl.kernel,
        out_type=jax.ShapeDtypeStruct((2 * _K, E), jnp.float32),
        mesh=mesh,
        compiler_params=pltpu.CompilerParams(needs_layout_passes=False),
        scratch_types=[
            pltpu.VMEM((_KH, n), jnp.float32),
            pltpu.VMEM((C,), jnp.int32),
            pltpu.VMEM((_KH, C), jnp.float32),
        ],
    )
    def sc_gather(srcdst_hbm, lt_hbm, out_hbm, tab, idx, out_v):
        wid = lax.axis_index("s") * NC + lax.axis_index("c")
        g = wid // 4
        h = (wid >> 1) & 1
        t = wid & 1
        row0 = t * _K + h * _KH
        pltpu.sync_copy(lt_hbm.at[pl.ds(row0, _KH), :], tab)
        my_chunks = (n_chunks - g + _NG - 1) // _NG

        def chunk_body(c, carry):
            base = (c * _NG + g) * C
            pltpu.sync_copy(srcdst_hbm.at[pl.ds(t * E + base, C)], idx)

            def vec_body(v, carry2):
                o = v * 16
                e = idx[pl.ds(o, 16)]
                for k in range(_KH):
                    ks = jnp.full((16,), k, jnp.int32)
                    out_v[k, pl.ds(o, 16)] = plsc.load_gather(tab, [ks, e])
                return carry2

            lax.fori_loop(0, C // 16, vec_body, 0)
            pltpu.sync_copy(out_v,
                            out_hbm.at[pl.ds(row0, _KH), pl.ds(base, C)])
            return carry

        lax.fori_loop(0, my_chunks, chunk_body, 0)

    return sc_gather


def _combine_body(logits_ref, anchors_ref, out_ref):
    l = logits_ref[0:_K, :] + logits_ref[_K:2 * _K, :]
    l = jnp.where(l >= 0, l, 0.01 * l)
    m = jnp.max(l, axis=0, keepdims=True)
    e = jnp.exp(l - m)
    s = jnp.sum(e, axis=0, keepdims=True)
    out_ref[:] = lax.dot_general(e / s, anchors_ref[:],
                                 (((0,), (0,)), ((), ())),
                                 preferred_element_type=jnp.float32, precision=lax.Precision.HIGHEST)


def _combine(logits_t, anchors, block_e):
    E = logits_t.shape[1]
    grid = E // block_e
    return pl.pallas_call(
        _combine_body,
        grid=(grid,),
        in_specs=[
            pl.BlockSpec((2 * _K, block_e), lambda i: (0, i)),
            pl.BlockSpec((_K, _D), lambda i: (0, 0)),
        ],
        out_specs=pl.BlockSpec((block_e, _D), lambda i: (i, 0)),
        out_shape=jax.ShapeDtypeStruct((E, _D), jnp.float32),
    )(logits_t, anchors)


def kernel(x_user, x_item, edge_index, W_proj_user, b_proj_user,
           W_proj_item, b_proj_item, W_score, b_score, anchors):
    E = edge_index.shape[1]
    srcdst = edge_index.reshape(2 * E)
    lt = _compute_tables(x_user, x_item, W_proj_user, b_proj_user,
                         W_proj_item, b_proj_item,
                         W_score[:_D], W_score[_D:], b_score)
    logits_t = _make_sc_gather(x_user.shape[0], E, 2560)(srcdst, lt)
    return _combine(logits_t, anchors, 2560)


# speedup vs baseline: 5.6497x; 1.0699x over previous
import functools

import jax
import jax.numpy as jnp
from jax import lax
from jax.experimental import pallas as pl
from jax.experimental.pallas import tpu as pltpu
from jax.experimental.pallas import tpu_sc as plsc

_D = 128
_K = 16


def _tables_body(xu_ref, xi_ref, wpu_ref, bpu_ref, wpi_ref, bpi_ref,
                 wsu_ref, wsi_ref, bs_ref, lt_ref):
    dn = (((0,), (1,)), ((), ()))
    wu = jnp.dot(wpu_ref[:], wsu_ref[:], preferred_element_type=jnp.float32, precision=lax.Precision.HIGHEST)
    cu = jnp.dot(bpu_ref[:], wsu_ref[:], preferred_element_type=jnp.float32, precision=lax.Precision.HIGHEST)
    lt_ref[0:_K, :] = (lax.dot_general(wu, xu_ref[:], dn,
                                       preferred_element_type=jnp.float32, precision=lax.Precision.HIGHEST)
                       + cu.reshape(_K, 1))
    wi = jnp.dot(wpi_ref[:], wsi_ref[:], preferred_element_type=jnp.float32, precision=lax.Precision.HIGHEST)
    ci = (jnp.dot(bpi_ref[:], wsi_ref[:], preferred_element_type=jnp.float32, precision=lax.Precision.HIGHEST)
          + bs_ref[:])
    lt_ref[_K:2 * _K, :] = (lax.dot_general(wi, xi_ref[:], dn,
                                            preferred_element_type=jnp.float32, precision=lax.Precision.HIGHEST)
                            + ci.reshape(_K, 1))


def _compute_tables(x_user, x_item, wpu, bpu, wpi, bpi, wsu, wsi, bs):
    n = x_user.shape[0]
    assert x_item.shape[0] == n
    return pl.pallas_call(
        _tables_body,
        out_shape=jax.ShapeDtypeStruct((2 * _K, n), jnp.float32),
    )(x_user, x_item, wpu, bpu.reshape(1, _D), wpi, bpi.reshape(1, _D),
      wsu, wsi, bs.reshape(1, _K))


_NG = 8
_KH = _K // 2


def _make_sc_gather(n, E, C):
    info = plsc.get_sparse_core_info()
    NC, NS = info.num_cores, info.num_subcores
    assert NC * NS == 32 and C % 128 == 0 and E % C == 0
    n_chunks = E // C
    mesh = plsc.VectorSubcoreMesh(core_axis_name="c", subcore_axis_name="s")

    @functools.partial(
        pl.kernel,
        out_type=jax.ShapeDtypeStruct((2 * _K, E), jnp.float32),
        mesh=mesh,
        compiler_params=pltpu.CompilerParams(needs_layout_passes=False),
        scratch_types=[
            pltpu.VMEM((_KH, n), jnp.float32),
            pltpu.VMEM((C,), jnp.int32),
            pltpu.VMEM((C,), jnp.int32),
            pltpu.VMEM((_KH, C), jnp.float32),
            pltpu.VMEM((_KH, C), jnp.float32),
            pltpu.SemaphoreType.DMA,
            pltpu.SemaphoreType.DMA,
            pltpu.SemaphoreType.DMA,
            pltpu.SemaphoreType.DMA,
        ],
    )
    def sc_gather(srcdst_hbm, lt_hbm, out_hbm, tab, idx0, idx1,
                  out0, out1, isem0, isem1, osem0, osem1):
        wid = lax.axis_index("s") * NC + lax.axis_index("c")
        g = wid // 4
        h = (wid >> 1) & 1
        t = wid & 1
        row0 = t * _K + h * _KH
        pltpu.sync_copy(lt_hbm.at[pl.ds(row0, _KH), :], tab)
        full_slots = n_chunks // _NG
        rem = n_chunks % _NG
        total_slots = full_slots + (1 if rem else 0)
        idx_bufs = (idx0, idx1)
        out_bufs = (out0, out1)
        isems = (isem0, isem1)
        osems = (osem0, osem1)

        def slot_base(s):
            return (s * _NG + g) * C

        def idx_copy(s):
            return pltpu.make_async_copy(
                srcdst_hbm.at[pl.ds(t * E + slot_base(s), C)],
                idx_bufs[s % 2], isems[s % 2])

        def out_copy(s):
            return pltpu.make_async_copy(
                out_bufs[s % 2],
                out_hbm.at[pl.ds(row0, _KH), pl.ds(slot_base(s), C)],
                osems[s % 2])

        def compute(s):
            idx = idx_bufs[s % 2]
            out_v = out_bufs[s % 2]

            def vec_body(v, carry):
                o = v * 16
                e = idx[pl.ds(o, 16)]
                for k in range(_KH):
                    ks = jnp.full((16,), k, jnp.int32)
                    out_v[k, pl.ds(o, 16)] = plsc.load_gather(tab, [ks, e])
                return carry

            lax.fori_loop(0, C // 16, vec_body, 0, unroll=4)

        def maybe_guard(s, fn):
            if s >= full_slots:
                pl.when(g < rem)(fn)
            else:
                fn()

        maybe_guard(0, lambda: idx_copy(0).start())
        for s in range(total_slots):
            def slot_work(s=s):
                if s + 1 < total_slots:
                    maybe_guard(s + 1, lambda: idx_copy(s + 1).start())
                idx_copy(s).wait()
                if s - 2 >= 0:
                    out_copy(s - 2).wait()
                compute(s)
                out_copy(s).start()

            maybe_guard(s, slot_work)

        if rem and full_slots >= 2:
            pl.when(g >= rem)(lambda: out_copy(full_slots - 2).wait())
        if full_slots >= 1:
            out_copy(full_slots - 1).wait()
        if rem:
            pl.when(g < rem)(lambda: out_copy(full_slots).wait())

    return sc_gather


def _combine_body(logits_ref, anchors_ref, out_ref):
    l = logits_ref[0:_K, :] + logits_ref[_K:2 * _K, :]
    l = jnp.where(l >= 0, l, 0.01 * l)
    m = jnp.max(l, axis=0, keepdims=True)
    e = jnp.exp(l - m)
    s = jnp.sum(e, axis=0, keepdims=True)
    out_ref[:] = lax.dot_general(e / s, anchors_ref[:],
                                 (((0,), (0,)), ((), ())),
                                 preferred_element_type=jnp.float32, precision=lax.Precision.HIGHEST)


def _combine(logits_t, anchors, block_e):
    E = logits_t.shape[1]
    grid = E // block_e
    return pl.pallas_call(
        _combine_body,
        grid=(grid,),
        in_specs=[
            pl.BlockSpec((2 * _K, block_e), lambda i: (0, i)),
            pl.BlockSpec((_K, _D), lambda i: (0, 0)),
        ],
        out_specs=pl.BlockSpec((block_e, _D), lambda i: (i, 0)),
        out_shape=jax.ShapeDtypeStruct((E, _D), jnp.float32),
    )(logits_t, anchors)


def kernel(x_user, x_item, edge_index, W_proj_user, b_proj_user,
           W_proj_item, b_proj_item, W_score, b_score, anchors):
    E = edge_index.shape[1]
    srcdst = edge_index.reshape(2 * E)
    lt = _compute_tables(x_user, x_item, W_proj_user, b_proj_user,
                         W_proj_item, b_proj_item,
                         W_score[:_D], W_score[_D:], b_score)
    logits_t = _make_sc_gather(x_user.shape[0], E, 2560)(srcdst, lt)
    return _combine(logits_t, anchors, 2560)


# speedup vs baseline: 9.1832x; 1.6254x over previous
import functools

import jax
import jax.numpy as jnp
from jax import lax
from jax.experimental import pallas as pl
from jax.experimental.pallas import tpu as pltpu
from jax.experimental.pallas import tpu_sc as plsc

_D = 128
_K = 16


def _tables_body(xu_ref, xi_ref, wpu_ref, bpu_ref, wpi_ref, bpi_ref,
                 wsu_ref, wsi_ref, bs_ref, lt_ref):
    dn = (((0,), (1,)), ((), ()))
    wu = jnp.dot(wpu_ref[:], wsu_ref[:], preferred_element_type=jnp.float32, precision=lax.Precision.HIGHEST)
    cu = jnp.dot(bpu_ref[:], wsu_ref[:], preferred_element_type=jnp.float32, precision=lax.Precision.HIGHEST)
    lt_ref[0:_K, :] = (lax.dot_general(wu, xu_ref[:], dn,
                                       preferred_element_type=jnp.float32, precision=lax.Precision.HIGHEST)
                       + cu.reshape(_K, 1))
    wi = jnp.dot(wpi_ref[:], wsi_ref[:], preferred_element_type=jnp.float32, precision=lax.Precision.HIGHEST)
    ci = (jnp.dot(bpi_ref[:], wsi_ref[:], preferred_element_type=jnp.float32, precision=lax.Precision.HIGHEST)
          + bs_ref[:])
    lt_ref[_K:2 * _K, :] = (lax.dot_general(wi, xi_ref[:], dn,
                                            preferred_element_type=jnp.float32, precision=lax.Precision.HIGHEST)
                            + ci.reshape(_K, 1))


def _compute_tables(x_user, x_item, wpu, bpu, wpi, bpi, wsu, wsi, bs):
    n = x_user.shape[0]
    assert x_item.shape[0] == n
    return pl.pallas_call(
        _tables_body,
        out_shape=jax.ShapeDtypeStruct((2 * _K, n), jnp.float32),
    )(x_user, x_item, wpu, bpu.reshape(1, _D), wpi, bpi.reshape(1, _D),
      wsu, wsi, bs.reshape(1, _K))


_NG = 8
_KH = _K // 2


def _make_sc_gather(n, E, C):
    info = plsc.get_sparse_core_info()
    NC, NS = info.num_cores, info.num_subcores
    assert NC * NS == 32 and C % 128 == 0 and E % C == 0
    n_chunks = E // C
    mesh = plsc.VectorSubcoreMesh(core_axis_name="c", subcore_axis_name="s")

    @functools.partial(
        pl.kernel,
        out_type=jax.ShapeDtypeStruct((2 * _K, E), jnp.float32),
        mesh=mesh,
        compiler_params=pltpu.CompilerParams(needs_layout_passes=False),
        scratch_types=[
            pltpu.VMEM((_KH, n), jnp.float32),
            pltpu.VMEM((C,), jnp.int32),
            pltpu.VMEM((C,), jnp.int32),
            pltpu.VMEM((_KH, C), jnp.float32),
            pltpu.VMEM((_KH, C), jnp.float32),
            pltpu.SemaphoreType.DMA,
            pltpu.SemaphoreType.DMA,
            pltpu.SemaphoreType.DMA,
            pltpu.SemaphoreType.DMA,
        ],
    )
    def sc_gather(srcdst_hbm, lt_hbm, out_hbm, tab, idx0, idx1,
                  out0, out1, isem0, isem1, osem0, osem1):
        wid = lax.axis_index("s") * NC + lax.axis_index("c")
        g = wid // 4
        h = (wid >> 1) & 1
        t = wid & 1
        row0 = t * _K + h * _KH
        pltpu.sync_copy(lt_hbm.at[pl.ds(row0, _KH), :], tab)
        full_slots = n_chunks // _NG
        rem = n_chunks % _NG
        total_slots = full_slots + (1 if rem else 0)
        idx_bufs = (idx0, idx1)
        out_bufs = (out0, out1)
        isems = (isem0, isem1)
        osems = (osem0, osem1)

        def slot_base(s):
            return (s * _NG + g) * C

        def idx_copy(s):
            return pltpu.make_async_copy(
                srcdst_hbm.at[pl.ds(t * E + slot_base(s), C)],
                idx_bufs[s % 2], isems[s % 2])

        def out_copy(s):
            return pltpu.make_async_copy(
                out_bufs[s % 2],
                out_hbm.at[pl.ds(row0, _KH), pl.ds(slot_base(s), C)],
                osems[s % 2])

        def compute(s):
            idx = idx_bufs[s % 2]
            out_v = out_bufs[s % 2]

            @plsc.parallel_loop(0, C, step=16, unroll=4)
            def vec_body(o):
                e = idx[pl.ds(o, 16)]
                for k in range(_KH):
                    ks = jnp.full((16,), k, jnp.int32)
                    out_v[k, pl.ds(o, 16)] = plsc.load_gather(tab, [ks, e])

        def maybe_guard(s, fn):
            if s >= full_slots:
                pl.when(g < rem)(fn)
            else:
                fn()

        maybe_guard(0, lambda: idx_copy(0).start())
        for s in range(total_slots):
            def slot_work(s=s):
                if s + 1 < total_slots:
                    maybe_guard(s + 1, lambda: idx_copy(s + 1).start())
                idx_copy(s).wait()
                if s - 2 >= 0:
                    out_copy(s - 2).wait()
                compute(s)
                out_copy(s).start()

            maybe_guard(s, slot_work)

        if rem and full_slots >= 2:
            pl.when(g >= rem)(lambda: out_copy(full_slots - 2).wait())
        if full_slots >= 1:
            out_copy(full_slots - 1).wait()
        if rem:
            pl.when(g < rem)(lambda: out_copy(full_slots).wait())

    return sc_gather


def _combine_body(logits_ref, anchors_ref, out_ref):
    l = logits_ref[0:_K, :] + logits_ref[_K:2 * _K, :]
    l = jnp.where(l >= 0, l, 0.01 * l)
    m = jnp.max(l, axis=0, keepdims=True)
    e = jnp.exp(l - m)
    s = jnp.sum(e, axis=0, keepdims=True)
    out_ref[:] = lax.dot_general(e / s, anchors_ref[:],
                                 (((0,), (0,)), ((), ())),
                                 preferred_element_type=jnp.float32)


def _combine(logits_t, anchors, block_e):
    E = logits_t.shape[1]
    grid = E // block_e
    return pl.pallas_call(
        _combine_body,
        grid=(grid,),
        in_specs=[
            pl.BlockSpec((2 * _K, block_e), lambda i: (0, i)),
            pl.BlockSpec((_K, _D), lambda i: (0, 0)),
        ],
        out_specs=pl.BlockSpec((block_e, _D), lambda i: (i, 0)),
        out_shape=jax.ShapeDtypeStruct((E, _D), jnp.float32),
    )(logits_t, anchors)


def kernel(x_user, x_item, edge_index, W_proj_user, b_proj_user,
           W_proj_item, b_proj_item, W_score, b_score, anchors):
    E = edge_index.shape[1]
    srcdst = edge_index.reshape(2 * E)
    lt = _compute_tables(x_user, x_item, W_proj_user, b_proj_user,
                         W_proj_item, b_proj_item,
                         W_score[:_D], W_score[_D:], b_score)
    logits_t = _make_sc_gather(x_user.shape[0], E, 2560)(srcdst, lt)
    return _combine(logits_t, anchors, 2560)
